# Initial kernel scaffold; baseline (speedup 1.0000x reference)
import jax
import jax.numpy as jnp
from jax.experimental import pallas as pl


def kernel(logits, k, p):
    raise NotImplementedError("write your pallas kernel here")


# speedup vs baseline: 20.2528x; 20.2528x over previous
import functools

import jax
import jax.numpy as jnp
import numpy as np
from jax import lax
from jax.experimental import pallas as pl
from jax.experimental.pallas import tpu as pltpu

_NEG_INF = float("-inf")


def _row_kernel(k_ref, p_ref, x_ref, q_ref, out_ref, samp_ref):
    row = pl.program_id(0)
    x = x_ref[0]
    S, L = x.shape

    b = lax.bitcast_convert_type(x, jnp.int32)
    key = jnp.where(b >= 0, b, b ^ jnp.int32(0x7FFFFFFF))
    kb = lax.bitcast_convert_type(key, jnp.uint32) ^ jnp.uint32(0x80000000)

    idx = (lax.broadcasted_iota(jnp.int32, (S, L), 0) * L
           + lax.broadcasted_iota(jnp.int32, (S, L), 1))

    M = jnp.max(x)
    kk = k_ref[row]
    pp = p_ref[row]

    def bit1(i, t):
        cand = t | (jnp.uint32(1) << (jnp.uint32(31) - i.astype(jnp.uint32)))
        c = jnp.sum(jnp.where(kb >= cand, jnp.int32(1), jnp.int32(0)))
        return jnp.where(c >= kk, cand, t)

    tk = lax.fori_loop(0, 32, bit1, jnp.uint32(0))

    e = jnp.exp(x - M)
    e_surv = jnp.where(kb >= tk, e, 0.0)
    s1 = jnp.sum(e_surv)
    target = pp * s1

    def bit2(i, t):
        bit = jnp.uint32(1) << (jnp.uint32(31) - i.astype(jnp.uint32))
        test = t | (bit - jnp.uint32(1))
        g = jnp.sum(jnp.where(kb > test, e_surv, 0.0))
        return jnp.where(g < target, t, t | bit)

    tp = lax.fori_loop(0, 32, bit2, jnp.uint32(0))

    tie = kb == tp
    e_star = jnp.sum(jnp.where(kb > tp, e_surv, 0.0))
    e_t = jnp.max(jnp.where(tie, e, 0.0))
    c_tie = jnp.sum(jnp.where(tie, jnp.int32(1), jnp.int32(0)))

    jj = (lax.broadcasted_iota(jnp.int32, (8, L), 0) * L
          + lax.broadcasted_iota(jnp.int32, (8, L), 1)).astype(jnp.float32)
    need = jnp.sum(jnp.where(jj * e_t + e_star < target,
                             jnp.int32(1), jnp.int32(0)))
    d = (target - e_star) / jnp.maximum(e_t, jnp.float32(1e-37))
    d = jnp.minimum(d, jnp.float32(2e9))
    fl = jnp.floor(d)
    need_ar = fl.astype(jnp.int32) + jnp.where(d > fl, jnp.int32(1), jnp.int32(0))
    need = jnp.where(need >= 8 * L, need_ar, need)
    need = jnp.minimum(need, c_tie)

    big = jnp.int32(1 << 30)
    ibits = int(S * L - 1).bit_length()

    def tie_bit(i, t):
        cand = t | (jnp.int32(1) << (jnp.int32(ibits - 1) - i))
        c = jnp.sum(jnp.where(tie & (idx >= cand), jnp.int32(1), jnp.int32(0)))
        return jnp.where(c >= need, cand, t)

    istar = lax.fori_loop(0, ibits, tie_bit, jnp.int32(0))

    kept = (kb > tp) | (tie & (idx >= istar))
    kmax = jnp.max(key)
    ilast = jnp.max(jnp.where(key == kmax, idx, jnp.int32(-1)))
    kept = kept | (idx == ilast)

    s_kept = jnp.sum(jnp.where(kept, e, 0.0))
    log_z = M + jnp.log(s_kept)
    out_ref[0] = jnp.where(kept, x - log_z, _NEG_INF)

    score = jnp.where(kept, (e / s_kept) / q_ref[0], -1.0)
    smax = jnp.max(score)
    samp = jnp.min(jnp.where(score == smax, idx, big))
    samp_ref[0] = jnp.broadcast_to(samp, (1, L))


@functools.partial(jax.jit, static_argnames=("interpret",))
def _run(logits, k, p, q_pad, interpret=False):
    bsz, vocab = logits.shape
    pv = ((vocab + 1023) // 1024) * 1024
    sub = pv // 128
    xp = jnp.pad(logits, ((0, 0), (0, pv - vocab)),
                 constant_values=_NEG_INF).reshape(bsz, sub, 128)

    out, samp = pl.pallas_call(
        _row_kernel,
        grid=(bsz,),
        in_specs=[
            pl.BlockSpec(memory_space=pltpu.SMEM),
            pl.BlockSpec(memory_space=pltpu.SMEM),
            pl.BlockSpec((1, sub, 128), lambda i: (i, 0, 0)),
            pl.BlockSpec((1, sub, 128), lambda i: (i, 0, 0)),
        ],
        out_specs=[
            pl.BlockSpec((1, sub, 128), lambda i: (i, 0, 0)),
            pl.BlockSpec((1, 1, 128), lambda i: (i, 0, 0)),
        ],
        out_shape=[
            jax.ShapeDtypeStruct((bsz, sub, 128), jnp.float32),
            jax.ShapeDtypeStruct((bsz, 1, 128), jnp.int32),
        ],
        interpret=interpret,
    )(k, p, xp, q_pad)

    logprobs = out.reshape(bsz, pv)[:, :vocab]
    samples = samp[:, 0, 0]
    return samples, logprobs


def kernel(logits, k, p):
    bsz, vocab = logits.shape
    pv = ((vocab + 1023) // 1024) * 1024
    sub = pv // 128
    q = jax.random.exponential(jax.random.key(42), (bsz, vocab), dtype=jnp.float32)
    q_pad = jnp.pad(q, ((0, 0), (0, pv - vocab)),
                    constant_values=1.0).reshape(bsz, sub, 128)
    return _run(logits, k.astype(jnp.int32), p, q_pad)


# speedup vs baseline: 60.4685x; 2.9857x over previous
import functools

import jax
import jax.numpy as jnp
import numpy as np
from jax import lax
from jax.experimental import pallas as pl
from jax.experimental.pallas import tpu as pltpu

_NEG_INF = float("-inf")
_R = 8


def _row_kernel(x_ref, q_ref, kp_ref, out_ref, samp_ref):
    x = x_ref[0]
    R, S, L = x.shape

    b = lax.bitcast_convert_type(x, jnp.int32)
    key = jnp.where(b >= 0, b, b ^ jnp.int32(0x7FFFFFFF))
    kb = lax.bitcast_convert_type(key, jnp.uint32) ^ jnp.uint32(0x80000000)

    idx = (lax.broadcasted_iota(jnp.int32, (R, S, L), 1) * L
           + lax.broadcasted_iota(jnp.int32, (R, S, L), 2))

    def rsum(v):
        return jnp.sum(v, axis=(1, 2), keepdims=True)

    def rmax(v):
        return jnp.max(v, axis=(1, 2), keepdims=True)

    M = rmax(x)
    kpv = kp_ref[0]
    kk = kpv[:, 0:1].astype(jnp.int32).reshape(R, 1, 1)
    pp = kpv[:, 1:2].reshape(R, 1, 1)

    one = jnp.int32(1)
    zero = jnp.int32(0)

    def bit1(i, t):
        cand = t | (jnp.uint32(1) << (jnp.uint32(31) - i.astype(jnp.uint32)))
        c = rsum(jnp.where(kb >= cand, one, zero))
        return jnp.where(c >= kk, cand, t)

    tk = lax.fori_loop(0, 32, bit1, jnp.zeros((R, 1, 1), jnp.uint32))

    e = jnp.exp(x - M)
    e_surv = jnp.where(kb >= tk, e, 0.0)
    s1 = rsum(e_surv)
    target = pp * s1

    def bit2(i, t):
        bit = jnp.uint32(1) << (jnp.uint32(31) - i.astype(jnp.uint32))
        test = t | (bit - jnp.uint32(1))
        g = rsum(jnp.where(kb > test, e_surv, 0.0))
        return jnp.where(g < target, t, t | bit)

    tp = lax.fori_loop(0, 32, bit2, jnp.zeros((R, 1, 1), jnp.uint32))

    tie = kb == tp
    e_star = rsum(jnp.where(kb > tp, e_surv, 0.0))
    e_t = rmax(jnp.where(tie, e, 0.0))
    c_tie = rsum(jnp.where(tie, one, zero))

    jj = (lax.broadcasted_iota(jnp.int32, (1, 8, L), 1) * L
          + lax.broadcasted_iota(jnp.int32, (1, 8, L), 2)).astype(jnp.float32)
    need = jnp.sum(jnp.where(jj * e_t + e_star < target, one, zero),
                   axis=(1, 2), keepdims=True)
    d = (target - e_star) / jnp.maximum(e_t, jnp.float32(1e-37))
    d = jnp.minimum(d, jnp.float32(2e9))
    fl = jnp.floor(d)
    need_ar = fl.astype(jnp.int32) + jnp.where(d > fl, one, zero)
    need = jnp.where(need >= 8 * L, need_ar, need)
    need = jnp.minimum(need, c_tie)

    big = jnp.int32(1 << 30)
    ibits = int(S * L - 1).bit_length()

    def tie_bit(i, t):
        cand = t | (one << (jnp.int32(ibits - 1) - i))
        c = rsum(jnp.where(tie & (idx >= cand), one, zero))
        return jnp.where(c >= need, cand, t)

    istar = lax.fori_loop(0, ibits, tie_bit, jnp.zeros((R, 1, 1), jnp.int32))

    kept = (kb > tp) | (tie & (idx >= istar))
    kmax = rmax(key)
    ilast = rmax(jnp.where(key == kmax, idx, jnp.int32(-1)))
    kept = kept | (idx == ilast)

    s_kept = rsum(jnp.where(kept, e, 0.0))
    log_z = M + jnp.log(s_kept)
    out_ref[0] = jnp.where(kept, x - log_z, _NEG_INF)

    score = jnp.where(kept, (e / s_kept) / q_ref[0], -1.0)
    smax = rmax(score)
    cand_idx = jnp.where(score == smax, idx, big)
    samp_lane = jnp.min(cand_idx, axis=1)
    samp_row = jnp.min(samp_lane, axis=1, keepdims=True)
    samp_ref[0] = jnp.broadcast_to(samp_row, (R, L))


@functools.partial(jax.jit, static_argnames=("interpret",))
def _run(logits, k, p, q_pad, interpret=False):
    bsz, vocab = logits.shape
    pv = ((vocab + 1023) // 1024) * 1024
    sub = pv // 128
    ng = bsz // _R
    xp = jnp.pad(logits, ((0, 0), (0, pv - vocab)),
                 constant_values=_NEG_INF).reshape(ng, _R, sub, 128)

    kp = jnp.stack([k.astype(jnp.float32), p], axis=-1).reshape(ng, _R, 2)

    out, samp = pl.pallas_call(
        _row_kernel,
        grid=(ng,),
        in_specs=[
            pl.BlockSpec((1, _R, sub, 128), lambda i: (i, 0, 0, 0)),
            pl.BlockSpec((1, _R, sub, 128), lambda i: (i, 0, 0, 0)),
            pl.BlockSpec((1, _R, 2), lambda i: (i, 0, 0)),
        ],
        out_specs=[
            pl.BlockSpec((1, _R, sub, 128), lambda i: (i, 0, 0, 0)),
            pl.BlockSpec((1, _R, 128), lambda i: (i, 0, 0)),
        ],
        out_shape=[
            jax.ShapeDtypeStruct((ng, _R, sub, 128), jnp.float32),
            jax.ShapeDtypeStruct((ng, _R, 128), jnp.int32),
        ],
        interpret=interpret,
    )(xp, q_pad, kp)

    logprobs = out.reshape(bsz, pv)[:, :vocab]
    samples = samp[:, :, 0].reshape(bsz)
    return samples, logprobs


_q_cache = {}


def kernel(logits, k, p):
    bsz, vocab = logits.shape
    pv = ((vocab + 1023) // 1024) * 1024
    sub = pv // 128
    ng = bsz // _R
    if (bsz, vocab) not in _q_cache:
        with jax.ensure_compile_time_eval():
            q = jax.random.exponential(jax.random.key(42), (bsz, vocab),
                                       dtype=jnp.float32)
            qp = jnp.pad(q, ((0, 0), (0, pv - vocab)), constant_values=1.0)
            _q_cache[(bsz, vocab)] = qp.reshape(ng, _R, sub, 128)
    q_pad = _q_cache[(bsz, vocab)]
    return _run(logits, k.astype(jnp.int32), p, q_pad)
